# Initial kernel scaffold; baseline (speedup 1.0000x reference)
import jax
import jax.numpy as jnp
from jax.experimental import pallas as pl


def kernel(x, edge_index, batch, edge_attr, params):
    raise NotImplementedError("write your pallas kernel here")


# speedup vs baseline: 2.5286x; 2.5286x over previous
import functools

import jax
import jax.numpy as jnp
from jax import lax
from jax.experimental import pallas as pl
from jax.experimental.pallas import tpu as pltpu
from jax.experimental.pallas import tpu_sc as plsc

N = 10000
E = 320000
HID = 128
EDGE_DIM = 16
NUM_GRAPHS = 64
LAYERS = 4

NC = 2
NS = 16
NW = NC * NS
EPT = E // NW
CH = 80
NCHUNK = EPT // CH
ZCH = 80
NZ = N // ZCH

f32 = jnp.float32


_EB = 2000


def _ee_body(ea_ref, w_ref, b_ref, o0, o1, o2, o3):
    ea = ea_ref[...]
    outs = (o0, o1, o2, o3)
    for l in range(LAYERS):
        outs[l][...] = (
            jnp.dot(ea, w_ref[l], preferred_element_type=f32) + b_ref[l]
        )


def _ee_call(edge_attr, w_stack, b_stack):
    grid = E // _EB
    return pl.pallas_call(
        _ee_body,
        grid=(grid,),
        in_specs=[
            pl.BlockSpec((_EB, EDGE_DIM), lambda i: (i, 0)),
            pl.BlockSpec((LAYERS, EDGE_DIM, HID), lambda i: (0, 0, 0)),
            pl.BlockSpec((LAYERS, 1, HID), lambda i: (0, 0, 0)),
        ],
        out_specs=[pl.BlockSpec((_EB, HID), lambda i: (i, 0))] * LAYERS,
        out_shape=[jax.ShapeDtypeStruct((E, HID), f32)] * LAYERS,
    )(edge_attr, w_stack, b_stack)


_sc_mesh = plsc.VectorSubcoreMesh(core_axis_name="c", subcore_axis_name="s")


@functools.partial(
    pl.kernel,
    out_type=jax.ShapeDtypeStruct((NC, N, HID), f32),
    mesh=_sc_mesh,
    scratch_types=[
        pltpu.VMEM((CH,), jnp.int32),
        pltpu.VMEM((CH,), jnp.int32),
        pltpu.VMEM((CH, HID), f32),
        pltpu.VMEM((CH, HID), f32),
        pltpu.VMEM_SHARED((N, HID), f32),
        pltpu.SemaphoreType.DMA,
    ],
)
def _sc_aggr(h_hbm, ee_hbm, src_hbm, dst_hbm, out_hbm,
             src_v, dst_v, rows_v, ee_v, aggr_sh, sem):
    c = lax.axis_index("c")
    s = lax.axis_index("s")
    wid = s * NC + c

    zvec = jnp.zeros((16,), f32)

    def zrow(r, _):
        for cc in range(HID // 16):
            rows_v[r, pl.ds(cc * 16, 16)] = zvec
        return 0

    lax.fori_loop(0, CH, zrow, 0)

    def zcp(i, _):
        k = s + NS * i
        @pl.when(k < NZ)
        def _():
            pltpu.sync_copy(rows_v, aggr_sh.at[pl.ds(k * ZCH, ZCH)])
        return 0

    lax.fori_loop(0, (NZ + NS - 1) // NS, zcp, 0)
    plsc.subcore_barrier()

    ebase = wid * EPT

    def chunk(j, _):
        off = ebase + j * CH
        pltpu.sync_copy(src_hbm.at[pl.ds(off, CH)], src_v)
        pltpu.sync_copy(dst_hbm.at[pl.ds(off, CH)], dst_v)
        pltpu.sync_copy(ee_hbm.at[pl.ds(off, CH)], ee_v)
        pltpu.async_copy(h_hbm.at[src_v], rows_v, sem).wait()

        def rbody(r, _):
            for cc in range(HID // 16):
                sl = pl.ds(cc * 16, 16)
                rows_v[r, sl] = jnp.maximum(rows_v[r, sl] + ee_v[r, sl], 0.0)
            return 0

        lax.fori_loop(0, CH, rbody, 0)
        pltpu.sync_copy(rows_v, aggr_sh.at[dst_v], add=True)
        return 0

    lax.fori_loop(0, NCHUNK, chunk, 0)
    plsc.subcore_barrier()

    def wb(i, _):
        k = s + NS * i
        @pl.when(k < NZ)
        def _():
            pltpu.sync_copy(aggr_sh.at[pl.ds(k * ZCH, ZCH)],
                            out_hbm.at[c, pl.ds(k * ZCH, ZCH)])
        return 0

    lax.fori_loop(0, (NZ + NS - 1) // NS, wb, 0)


_NB = 1000


def _mlp_body(h_ref, p_ref, w_ref, b_ref, o_ref):
    z = h_ref[...] + p_ref[0] + p_ref[1]
    z = jnp.maximum(jnp.dot(z, w_ref[0], preferred_element_type=f32) + b_ref[0], 0.0)
    z = jnp.maximum(jnp.dot(z, w_ref[1], preferred_element_type=f32) + b_ref[1], 0.0)
    z = jnp.dot(z, w_ref[2], preferred_element_type=f32) + b_ref[2]
    o_ref[...] = jnp.maximum(z, 0.0)


def _mlp_call(h, parts, w_stack, b_stack):
    grid = N // _NB
    return pl.pallas_call(
        _mlp_body,
        grid=(grid,),
        in_specs=[
            pl.BlockSpec((_NB, HID), lambda i: (i, 0)),
            pl.BlockSpec((NC, _NB, HID), lambda i: (0, i, 0)),
            pl.BlockSpec((3, HID, HID), lambda i: (0, 0, 0)),
            pl.BlockSpec((3, 1, HID), lambda i: (0, 0, 0)),
        ],
        out_specs=pl.BlockSpec((_NB, HID), lambda i: (i, 0)),
        out_shape=jax.ShapeDtypeStruct((N, HID), f32),
    )(h, parts, w_stack, b_stack)


def _pool_body(h_ref, batch_ref, w1_ref, b1_ref, w2_ref, b2_ref, o_ref):
    h = h_ref[...]
    b = batch_ref[...]
    gid = lax.broadcasted_iota(jnp.int32, (NUM_GRAPHS, N), 0)
    onehot = (b == gid).astype(f32)
    sums = jnp.dot(onehot, h, preferred_element_type=f32)
    cnt = jnp.sum(onehot, axis=1, keepdims=True)
    hg = sums / jnp.maximum(cnt, 1.0)
    t = jnp.maximum(jnp.dot(hg, w1_ref[...], preferred_element_type=f32)
                    + b1_ref[...], 0.0)
    o_ref[...] = jnp.dot(t, w2_ref[...], preferred_element_type=f32) + b2_ref[...]


def _pool_call(h, batch2d, w1, b1, w2, b2):
    return pl.pallas_call(
        _pool_body,
        out_shape=jax.ShapeDtypeStruct((NUM_GRAPHS, 1), f32),
    )(h, batch2d, w1, b1, w2, b2)


def kernel(x, edge_index, batch, edge_attr, params):
    src = edge_index[0]
    dst = edge_index[1]

    convs = params["convs"]
    we = jnp.stack([p["We"] for p in convs])
    be = jnp.stack([p["be"].reshape(1, HID) for p in convs])
    ees = _ee_call(edge_attr, we, be)

    h = x
    for l in range(LAYERS):
        p = convs[l]
        parts = _sc_aggr(h, ees[l], src, dst)
        wl = jnp.stack([p["W1"], p["W2"], p["W3"]])
        bl = jnp.stack([p["b1"].reshape(1, HID),
                        p["b2"].reshape(1, HID),
                        p["b3"].reshape(1, HID)])
        h = _mlp_call(h, parts, wl, bl)

    hp = params["head"]
    out = _pool_call(h, batch.reshape(1, N),
                     hp["W1"], hp["b1"].reshape(1, HID),
                     hp["W2"], hp["b2"].reshape(1, 1))
    return out.reshape(-1)


# speedup vs baseline: 5.1872x; 2.0514x over previous
import functools

import jax
import jax.numpy as jnp
from jax import lax
from jax.experimental import pallas as pl
from jax.experimental.pallas import tpu as pltpu
from jax.experimental.pallas import tpu_sc as plsc

N = 10000
E = 320000
HID = 128
EDGE_DIM = 16
NUM_GRAPHS = 64
LAYERS = 4

NC = 2
NS = 16
NW = NC * NS
EPT = E // NW
CH = 40
NCHUNK = EPT // CH
ZCH = 40
NZ = N // ZCH
NRB = 4
NEB = 2

f32 = jnp.float32


_EB = 2000


def _ee_body(ea_ref, w_ref, b_ref, o0, o1, o2, o3):
    ea = ea_ref[...]
    outs = (o0, o1, o2, o3)
    for l in range(LAYERS):
        outs[l][...] = (
            jnp.dot(ea, w_ref[l], preferred_element_type=f32) + b_ref[l]
        )


def _ee_call(edge_attr, w_stack, b_stack):
    grid = E // _EB
    return pl.pallas_call(
        _ee_body,
        grid=(grid,),
        in_specs=[
            pl.BlockSpec((_EB, EDGE_DIM), lambda i: (i, 0)),
            pl.BlockSpec((LAYERS, EDGE_DIM, HID), lambda i: (0, 0, 0)),
            pl.BlockSpec((LAYERS, 1, HID), lambda i: (0, 0, 0)),
        ],
        out_specs=[pl.BlockSpec((_EB, HID), lambda i: (i, 0))] * LAYERS,
        out_shape=[jax.ShapeDtypeStruct((E, HID), f32)] * LAYERS,
    )(edge_attr, w_stack, b_stack)


_sc_mesh = plsc.VectorSubcoreMesh(core_axis_name="c", subcore_axis_name="s")


@functools.partial(
    pl.kernel,
    out_type=jax.ShapeDtypeStruct((NC, N, HID), f32),
    mesh=_sc_mesh,
    scratch_types=[
        pltpu.VMEM((EPT,), jnp.int32),
        [pltpu.VMEM((CH,), jnp.int32) for _ in range(NRB)],
        [pltpu.VMEM((CH, HID), f32) for _ in range(NRB)],
        [pltpu.VMEM((CH, HID), f32) for _ in range(NEB)],
        pltpu.VMEM_SHARED((N, HID), f32),
        [pltpu.SemaphoreType.DMA for _ in range(NRB)],
        [pltpu.SemaphoreType.DMA for _ in range(NEB)],
        [pltpu.SemaphoreType.DMA for _ in range(NRB)],
        [pltpu.SemaphoreType.DMA for _ in range(NRB)],
    ],
)
def _sc_aggr(h_hbm, ee_hbm, src_hbm, dst_hbm, out_hbm,
             src_all, dst_v, rows_v, ee_v, aggr_sh, sg, se, sd, ss):
    c = lax.axis_index("c")
    s = lax.axis_index("s")
    wid = s * NC + c
    ebase = wid * EPT

    zvec = jnp.zeros((16,), f32)

    def zrow(r, _):
        for cc in range(HID // 16):
            rows_v[0][r, pl.ds(cc * 16, 16)] = zvec
        return 0

    lax.fori_loop(0, CH, zrow, 0)

    def zcp(i, _):
        k = s + NS * i
        @pl.when(k < NZ)
        def _():
            pltpu.sync_copy(rows_v[0], aggr_sh.at[pl.ds(k * ZCH, ZCH)])
        return 0

    lax.fori_loop(0, (NZ + NS - 1) // NS, zcp, 0)
    plsc.subcore_barrier()

    pltpu.sync_copy(src_hbm.at[pl.ds(ebase, EPT)], src_all)

    def issue(j, br, be):
        off = ebase + j * CH
        pltpu.async_copy(dst_hbm.at[pl.ds(off, CH)], dst_v[br], sd[br])
        pltpu.async_copy(ee_hbm.at[pl.ds(off, CH)], ee_v[be], se[be])
        pltpu.async_copy(h_hbm.at[src_all.at[pl.ds(j * CH, CH)]],
                         rows_v[br], sg[br])

    for p in range(NEB):
        issue(p, p, p)

    def step(jj, _):
        for br in range(NRB):
            j = jj * NRB + br
            be = br % NEB

            @pl.when(j < NCHUNK)
            def _():
                pltpu.make_async_copy(dst_hbm.at[pl.ds(0, CH)],
                                      dst_v[br], sd[br]).wait()
                pltpu.make_async_copy(ee_hbm.at[pl.ds(0, CH)],
                                      ee_v[be], se[be]).wait()
                pltpu.make_async_copy(ee_hbm.at[pl.ds(0, CH)],
                                      rows_v[br], sg[br]).wait()

                def rbody(r, _):
                    for cc in range(HID // 16):
                        sl = pl.ds(cc * 16, 16)
                        rows_v[br][r, sl] = jnp.maximum(
                            rows_v[br][r, sl] + ee_v[be][r, sl], 0.0)
                    return 0

                lax.fori_loop(0, CH, rbody, 0)
                pltpu.async_copy(rows_v[br], aggr_sh.at[dst_v[br]], ss[br],
                                 add=True)

                @pl.when(j + NEB < NCHUNK)
                def _():
                    nbr = (br + NEB) % NRB
                    @pl.when(j + NEB >= NRB)
                    def _():
                        pltpu.make_async_copy(rows_v[nbr],
                                              aggr_sh.at[pl.ds(0, CH)],
                                              ss[nbr]).wait()
                    issue(j + NEB, nbr, be)
        return 0

    lax.fori_loop(0, (NCHUNK + NRB - 1) // NRB, step, 0)

    for br in range(NRB):
        pltpu.make_async_copy(rows_v[br], aggr_sh.at[pl.ds(0, CH)],
                              ss[br]).wait()
    plsc.subcore_barrier()

    def wb(i, _):
        k = s + NS * i
        @pl.when(k < NZ)
        def _():
            pltpu.sync_copy(aggr_sh.at[pl.ds(k * ZCH, ZCH)],
                            out_hbm.at[c, pl.ds(k * ZCH, ZCH)])
        return 0

    lax.fori_loop(0, (NZ + NS - 1) // NS, wb, 0)


_NB = 1000


def _mlp_body(h_ref, p_ref, w_ref, b_ref, o_ref):
    z = h_ref[...] + p_ref[0] + p_ref[1]
    z = jnp.maximum(jnp.dot(z, w_ref[0], preferred_element_type=f32) + b_ref[0], 0.0)
    z = jnp.maximum(jnp.dot(z, w_ref[1], preferred_element_type=f32) + b_ref[1], 0.0)
    z = jnp.dot(z, w_ref[2], preferred_element_type=f32) + b_ref[2]
    o_ref[...] = jnp.maximum(z, 0.0)


def _mlp_call(h, parts, w_stack, b_stack):
    grid = N // _NB
    return pl.pallas_call(
        _mlp_body,
        grid=(grid,),
        in_specs=[
            pl.BlockSpec((_NB, HID), lambda i: (i, 0)),
            pl.BlockSpec((NC, _NB, HID), lambda i: (0, i, 0)),
            pl.BlockSpec((3, HID, HID), lambda i: (0, 0, 0)),
            pl.BlockSpec((3, 1, HID), lambda i: (0, 0, 0)),
        ],
        out_specs=pl.BlockSpec((_NB, HID), lambda i: (i, 0)),
        out_shape=jax.ShapeDtypeStruct((N, HID), f32),
    )(h, parts, w_stack, b_stack)


def _pool_body(h_ref, batch_ref, w1_ref, b1_ref, w2_ref, b2_ref, o_ref):
    h = h_ref[...]
    b = batch_ref[...]
    gid = lax.broadcasted_iota(jnp.int32, (NUM_GRAPHS, N), 0)
    onehot = (b == gid).astype(f32)
    sums = jnp.dot(onehot, h, preferred_element_type=f32)
    cnt = jnp.sum(onehot, axis=1, keepdims=True)
    hg = sums / jnp.maximum(cnt, 1.0)
    t = jnp.maximum(jnp.dot(hg, w1_ref[...], preferred_element_type=f32)
                    + b1_ref[...], 0.0)
    o_ref[...] = jnp.dot(t, w2_ref[...], preferred_element_type=f32) + b2_ref[...]


def _pool_call(h, batch2d, w1, b1, w2, b2):
    return pl.pallas_call(
        _pool_body,
        out_shape=jax.ShapeDtypeStruct((NUM_GRAPHS, 1), f32),
    )(h, batch2d, w1, b1, w2, b2)


def kernel(x, edge_index, batch, edge_attr, params):
    src = edge_index[0]
    dst = edge_index[1]

    convs = params["convs"]
    we = jnp.stack([p["We"] for p in convs])
    be = jnp.stack([p["be"].reshape(1, HID) for p in convs])
    ees = _ee_call(edge_attr, we, be)

    h = x
    for l in range(LAYERS):
        p = convs[l]
        parts = _sc_aggr(h, ees[l], src, dst)
        wl = jnp.stack([p["W1"], p["W2"], p["W3"]])
        bl = jnp.stack([p["b1"].reshape(1, HID),
                        p["b2"].reshape(1, HID),
                        p["b3"].reshape(1, HID)])
        h = _mlp_call(h, parts, wl, bl)

    hp = params["head"]
    out = _pool_call(h, batch.reshape(1, N),
                     hp["W1"], hp["b1"].reshape(1, HID),
                     hp["W2"], hp["b2"].reshape(1, 1))
    return out.reshape(-1)


# speedup vs baseline: 5.3812x; 1.0374x over previous
import functools

import jax
import jax.numpy as jnp
from jax import lax
from jax.experimental import pallas as pl
from jax.experimental.pallas import tpu as pltpu
from jax.experimental.pallas import tpu_sc as plsc

N = 10000
E = 320000
HID = 128
EDGE_DIM = 16
NUM_GRAPHS = 64
LAYERS = 4

NC = 2
NS = 16
NW = NC * NS
EPT = E // NW
CH = 40
NCHUNK = EPT // CH
ZCH = 40
NZ = N // ZCH
NRB = 4
NEB = 2

f32 = jnp.float32


_EB = 2000


def _ee_body(ea_ref, w_ref, b_ref, o0, o1, o2, o3):
    ea = ea_ref[...]
    outs = (o0, o1, o2, o3)
    for l in range(LAYERS):
        outs[l][...] = (
            jnp.dot(ea, w_ref[l], preferred_element_type=f32) + b_ref[l]
        )


def _ee_call(edge_attr, w_stack, b_stack):
    grid = E // _EB
    return pl.pallas_call(
        _ee_body,
        grid=(grid,),
        in_specs=[
            pl.BlockSpec((_EB, EDGE_DIM), lambda i: (i, 0)),
            pl.BlockSpec((LAYERS, EDGE_DIM, HID), lambda i: (0, 0, 0)),
            pl.BlockSpec((LAYERS, 1, HID), lambda i: (0, 0, 0)),
        ],
        out_specs=[pl.BlockSpec((_EB, HID), lambda i: (i, 0))] * LAYERS,
        out_shape=[jax.ShapeDtypeStruct((E, HID), f32)] * LAYERS,
    )(edge_attr, w_stack, b_stack)


_sc_mesh = plsc.VectorSubcoreMesh(core_axis_name="c", subcore_axis_name="s")


@functools.partial(
    pl.kernel,
    out_type=jax.ShapeDtypeStruct((NC, N, HID), f32),
    mesh=_sc_mesh,
    scratch_types=[
        pltpu.VMEM((EPT,), jnp.int32),
        [pltpu.VMEM((CH,), jnp.int32) for _ in range(NRB)],
        [pltpu.VMEM((CH, HID), f32) for _ in range(NRB)],
        [pltpu.VMEM((CH, HID), f32) for _ in range(NEB)],
        pltpu.VMEM_SHARED((N, HID), f32),
        [pltpu.SemaphoreType.DMA for _ in range(NRB)],
        [pltpu.SemaphoreType.DMA for _ in range(NEB)],
        [pltpu.SemaphoreType.DMA for _ in range(NRB)],
        [pltpu.SemaphoreType.DMA for _ in range(NRB)],
        pltpu.SemaphoreType.DMA,
    ],
)
def _sc_aggr(h_hbm, ee_hbm, src_hbm, dst_hbm, out_hbm,
             src_all, dst_v, rows_v, ee_v, aggr_sh, sg, se, sd, ss, szw):
    c = lax.axis_index("c")
    s = lax.axis_index("s")
    wid = s * NC + c
    ebase = wid * EPT

    zvec = jnp.zeros((16,), f32)

    def zrow(r, _):
        for cc in range(HID // 16):
            rows_v[0][r, pl.ds(cc * 16, 16)] = zvec
        return 0

    lax.fori_loop(0, CH, zrow, 0)

    def zcp(i, _):
        k = s + NS * i
        @pl.when(k < NZ)
        def _():
            pltpu.async_copy(rows_v[0], aggr_sh.at[pl.ds(k * ZCH, ZCH)], szw)
        return 0

    lax.fori_loop(0, (NZ + NS - 1) // NS, zcp, 0)

    def zwt(i, _):
        k = s + NS * i
        @pl.when(k < NZ)
        def _():
            pltpu.make_async_copy(rows_v[0], aggr_sh.at[pl.ds(0, ZCH)],
                                  szw).wait()
        return 0

    lax.fori_loop(0, (NZ + NS - 1) // NS, zwt, 0)
    plsc.subcore_barrier()

    pltpu.sync_copy(src_hbm.at[pl.ds(ebase, EPT)], src_all)

    def issue_g(j, br):
        off = ebase + j * CH
        pltpu.async_copy(dst_hbm.at[pl.ds(off, CH)], dst_v[br], sd[br])
        pltpu.async_copy(h_hbm.at[src_all.at[pl.ds(j * CH, CH)]],
                         rows_v[br], sg[br])

    def issue_e(j, be):
        off = ebase + j * CH
        pltpu.async_copy(ee_hbm.at[pl.ds(off, CH)], ee_v[be], se[be])

    for p in range(NEB):
        issue_g(p, p)
        issue_e(p, p)

    def step(jj, _):
        for br in range(NRB):
            j = jj * NRB + br
            be = br % NEB

            @pl.when(j < NCHUNK)
            def _():
                pltpu.make_async_copy(ee_hbm.at[pl.ds(0, CH)],
                                      rows_v[br], sg[br]).wait()

                @pl.when(j + NEB < NCHUNK)
                def _():
                    nbr = (br + NEB) % NRB
                    @pl.when(j + NEB >= NRB)
                    def _():
                        pltpu.make_async_copy(rows_v[nbr],
                                              aggr_sh.at[pl.ds(0, CH)],
                                              ss[nbr]).wait()
                    issue_g(j + NEB, nbr)

                pltpu.make_async_copy(ee_hbm.at[pl.ds(0, CH)],
                                      ee_v[be], se[be]).wait()

                def rbody(r, _):
                    for cc in range(HID // 16):
                        sl = pl.ds(cc * 16, 16)
                        rows_v[br][r, sl] = jnp.maximum(
                            rows_v[br][r, sl] + ee_v[be][r, sl], 0.0)
                    return 0

                lax.fori_loop(0, CH, rbody, 0)

                pltpu.make_async_copy(dst_hbm.at[pl.ds(0, CH)],
                                      dst_v[br], sd[br]).wait()
                pltpu.async_copy(rows_v[br], aggr_sh.at[dst_v[br]], ss[br],
                                 add=True)

                @pl.when(j + NEB < NCHUNK)
                def _():
                    issue_e(j + NEB, be)
        return 0

    lax.fori_loop(0, (NCHUNK + NRB - 1) // NRB, step, 0)

    for br in range(NRB):
        pltpu.make_async_copy(rows_v[br], aggr_sh.at[pl.ds(0, CH)],
                              ss[br]).wait()
    plsc.subcore_barrier()

    def wb(i, _):
        k = s + NS * i
        @pl.when(k < NZ)
        def _():
            pltpu.async_copy(aggr_sh.at[pl.ds(k * ZCH, ZCH)],
                             out_hbm.at[c, pl.ds(k * ZCH, ZCH)], szw)
        return 0

    lax.fori_loop(0, (NZ + NS - 1) // NS, wb, 0)

    def wbt(i, _):
        k = s + NS * i
        @pl.when(k < NZ)
        def _():
            pltpu.make_async_copy(aggr_sh.at[pl.ds(0, ZCH)],
                                  out_hbm.at[c, pl.ds(0, ZCH)], szw).wait()
        return 0

    lax.fori_loop(0, (NZ + NS - 1) // NS, wbt, 0)


_NB = 1000


def _mlp_body(h_ref, p_ref, w_ref, b_ref, o_ref):
    z = h_ref[...] + p_ref[0] + p_ref[1]
    z = jnp.maximum(jnp.dot(z, w_ref[0], preferred_element_type=f32) + b_ref[0], 0.0)
    z = jnp.maximum(jnp.dot(z, w_ref[1], preferred_element_type=f32) + b_ref[1], 0.0)
    z = jnp.dot(z, w_ref[2], preferred_element_type=f32) + b_ref[2]
    o_ref[...] = jnp.maximum(z, 0.0)


def _mlp_call(h, parts, w_stack, b_stack):
    grid = N // _NB
    return pl.pallas_call(
        _mlp_body,
        grid=(grid,),
        in_specs=[
            pl.BlockSpec((_NB, HID), lambda i: (i, 0)),
            pl.BlockSpec((NC, _NB, HID), lambda i: (0, i, 0)),
            pl.BlockSpec((3, HID, HID), lambda i: (0, 0, 0)),
            pl.BlockSpec((3, 1, HID), lambda i: (0, 0, 0)),
        ],
        out_specs=pl.BlockSpec((_NB, HID), lambda i: (i, 0)),
        out_shape=jax.ShapeDtypeStruct((N, HID), f32),
    )(h, parts, w_stack, b_stack)


def _pool_body(h_ref, batch_ref, w1_ref, b1_ref, w2_ref, b2_ref, o_ref):
    h = h_ref[...]
    b = batch_ref[...]
    gid = lax.broadcasted_iota(jnp.int32, (NUM_GRAPHS, N), 0)
    onehot = (b == gid).astype(f32)
    sums = jnp.dot(onehot, h, preferred_element_type=f32)
    cnt = jnp.sum(onehot, axis=1, keepdims=True)
    hg = sums / jnp.maximum(cnt, 1.0)
    t = jnp.maximum(jnp.dot(hg, w1_ref[...], preferred_element_type=f32)
                    + b1_ref[...], 0.0)
    o_ref[...] = jnp.dot(t, w2_ref[...], preferred_element_type=f32) + b2_ref[...]


def _pool_call(h, batch2d, w1, b1, w2, b2):
    return pl.pallas_call(
        _pool_body,
        out_shape=jax.ShapeDtypeStruct((NUM_GRAPHS, 1), f32),
    )(h, batch2d, w1, b1, w2, b2)


def kernel(x, edge_index, batch, edge_attr, params):
    src = edge_index[0]
    dst = edge_index[1]

    convs = params["convs"]
    we = jnp.stack([p["We"] for p in convs])
    be = jnp.stack([p["be"].reshape(1, HID) for p in convs])
    ees = _ee_call(edge_attr, we, be)

    h = x
    for l in range(LAYERS):
        p = convs[l]
        parts = _sc_aggr(h, ees[l], src, dst)
        wl = jnp.stack([p["W1"], p["W2"], p["W3"]])
        bl = jnp.stack([p["b1"].reshape(1, HID),
                        p["b2"].reshape(1, HID),
                        p["b3"].reshape(1, HID)])
        h = _mlp_call(h, parts, wl, bl)

    hp = params["head"]
    out = _pool_call(h, batch.reshape(1, N),
                     hp["W1"], hp["b1"].reshape(1, HID),
                     hp["W2"], hp["b2"].reshape(1, 1))
    return out.reshape(-1)
